# baseline (speedup 1.0000x reference)
import jax
import jax.numpy as jnp
from jax import lax
from jax.experimental import pallas as pl
from jax.experimental.pallas import tpu as pltpu
from jax.experimental.pallas import tpu_sc as plsc

_WIN = 128
_NW = 32


def _sc_gather_wide(x, idxT):
    K, Npc = idxT.shape
    D = x.shape[1]
    nwin = Npc // _WIN
    J = K * nwin
    Tmax = (J + _NW - 1) // _NW
    mesh = plsc.VectorSubcoreMesh(core_axis_name="core",
                                  subcore_axis_name="subcore")

    @pl.kernel(
        out_type=jax.ShapeDtypeStruct((Npc, K * D), x.dtype),
        mesh=mesh,
        scratch_types=[
            pltpu.VMEM((1, _WIN), jnp.int32),
            pltpu.VMEM((1, _WIN), jnp.int32),
            pltpu.VMEM((_WIN, D), x.dtype),
            pltpu.VMEM((_WIN, D), x.dtype),
            pltpu.SemaphoreType.DMA,
            pltpu.SemaphoreType.DMA,
            pltpu.SemaphoreType.DMA,
            pltpu.SemaphoreType.DMA,
        ],
    )
    def gather_kernel(x_hbm, i_hbm, o_hbm, idx0, idx1, rows0, rows1,
                      g0, g1, w0, w1):
        wid = lax.axis_index("core") * 16 + lax.axis_index("subcore")
        idx_bufs = (idx0, idx1)
        row_bufs = (rows0, rows1)
        gsem = (g0, g1)
        wsem = (w0, w1)

        def load_and_gather(s, j):
            i, k = j // K, j % K
            pltpu.sync_copy(i_hbm.at[pl.ds(k, 1), pl.ds(i * _WIN, _WIN)],
                            idx_bufs[s])
            pltpu.async_copy(x_hbm.at[idx_bufs[s].at[0]], row_bufs[s], gsem[s])

        def gather_wait(s):
            pltpu.make_async_copy(x_hbm.at[idx_bufs[s].at[0]], row_bufs[s],
                                  gsem[s]).wait()

        def _wb_slices(j):
            i, k = j // K, j % K
            return (pl.ds(i * _WIN, _WIN), pl.ds(k * D, D))

        def wb_start(s, j):
            pltpu.async_copy(row_bufs[s], o_hbm.at[_wb_slices(j)], wsem[s])

        def wb_wait(s, j):
            pltpu.make_async_copy(row_bufs[s], o_hbm.at[_wb_slices(j)],
                                  wsem[s]).wait()

        @pl.when(wid < J)
        def _():
            load_and_gather(0, wid)

        @pl.when(wid + _NW < J)
        def _():
            load_and_gather(1, wid + _NW)

        @pl.loop(0, (Tmax + 1) // 2)
        def _(t):
            jA = wid + _NW * (2 * t)
            jB = jA + _NW
            jC = jB + _NW
            jD = jC + _NW

            @pl.when(jA < J)
            def _():
                gather_wait(0)
                wb_start(0, jA)

            @pl.when(jB < J)
            def _():
                gather_wait(1)
                wb_start(1, jB)

            @pl.when(jC < J)
            def _():
                wb_wait(0, jA)
                load_and_gather(0, jC)

            @pl.when(jD < J)
            def _():
                wb_wait(1, jB)
                load_and_gather(1, jD)

        @pl.when(wid < J)
        def _():
            wb_wait(0, 0)

        @pl.when(wid + _NW < J)
        def _():
            wb_wait(1, 0)

    return gather_kernel(x, idxT)


def _tc_matmul_into(g, Wb, b2, acc, row_off, block_m):
    rows, K = g.shape
    F = Wb.shape[1]
    off_blk = row_off // block_m

    def mm_kernel(g_ref, w_ref, b_ref, acc_ref, o_ref):
        del acc_ref
        gb = g_ref[...].astype(jnp.bfloat16)
        o_ref[...] = (
            jnp.dot(gb, w_ref[...], preferred_element_type=jnp.float32)
            + b_ref[...]
        )

    return pl.pallas_call(
        mm_kernel,
        grid=(rows // block_m,),
        in_specs=[
            pl.BlockSpec((block_m, K), lambda i: (i, 0)),
            pl.BlockSpec((K, F), lambda i: (0, 0)),
            pl.BlockSpec((1, F), lambda i: (0, 0)),
            pl.BlockSpec(memory_space=pl.ANY),
        ],
        out_specs=pl.BlockSpec((block_m, F), lambda i: (off_blk + i, 0)),
        out_shape=jax.ShapeDtypeStruct(acc.shape, jnp.float32),
        input_output_aliases={3: 0},
    )(g, Wb, b2, acc)


def kernel(x, hex_in, W, b):
    N, D = x.shape
    Np = (N + 127) // 128 * 128
    idxT = jnp.pad(hex_in.astype(jnp.int32).T, ((0, 0), (0, Np - N)))
    Wb = W.astype(jnp.bfloat16)
    b2 = b.reshape(1, -1)
    bounds = [0, 308 * 128, Np]
    acc = jnp.zeros((Np, W.shape[1]), jnp.float32)
    prev_g = None
    for c in range(len(bounds) - 1):
        lo, hi = bounds[c], bounds[c + 1]
        idx_c = jax.lax.slice(idxT, (0, lo), (idxT.shape[0], hi))
        if prev_g is not None:
            idx_c, _ = jax.lax.optimization_barrier((idx_c, prev_g))
        g_c = _sc_gather_wide(x, idx_c)
        prev_g = g_c
        acc = _tc_matmul_into(g_c, Wb, b2, acc, lo, block_m=448)
    return acc[:N]


# speedup vs baseline: 1.0457x; 1.0457x over previous
import jax
import jax.numpy as jnp
from jax import lax
from jax.experimental import pallas as pl
from jax.experimental.pallas import tpu as pltpu
from jax.experimental.pallas import tpu_sc as plsc

_WIN = 128
_NW = 32


def _sc_gather_wide(x, idxT):
    K, Npc = idxT.shape
    D = x.shape[1]
    nwin = Npc // _WIN
    J = K * nwin
    Tmax = (J + _NW - 1) // _NW
    mesh = plsc.VectorSubcoreMesh(core_axis_name="core",
                                  subcore_axis_name="subcore")

    @pl.kernel(
        out_type=jax.ShapeDtypeStruct((Npc, K * D), x.dtype),
        mesh=mesh,
        scratch_types=[
            pltpu.VMEM((1, _WIN), jnp.int32),
            pltpu.VMEM((1, _WIN), jnp.int32),
            pltpu.VMEM((_WIN, D), x.dtype),
            pltpu.VMEM((_WIN, D), x.dtype),
            pltpu.SemaphoreType.DMA,
            pltpu.SemaphoreType.DMA,
            pltpu.SemaphoreType.DMA,
            pltpu.SemaphoreType.DMA,
        ],
    )
    def gather_kernel(x_hbm, i_hbm, o_hbm, idx0, idx1, rows0, rows1,
                      g0, g1, w0, w1):
        wid = lax.axis_index("core") * 16 + lax.axis_index("subcore")
        idx_bufs = (idx0, idx1)
        row_bufs = (rows0, rows1)
        gsem = (g0, g1)
        wsem = (w0, w1)

        def load_and_gather(s, j):
            i, k = j // K, j % K
            pltpu.sync_copy(i_hbm.at[pl.ds(k, 1), pl.ds(i * _WIN, _WIN)],
                            idx_bufs[s])
            pltpu.async_copy(x_hbm.at[idx_bufs[s].at[0]], row_bufs[s], gsem[s])

        def gather_wait(s):
            pltpu.make_async_copy(x_hbm.at[idx_bufs[s].at[0]], row_bufs[s],
                                  gsem[s]).wait()

        def _wb_slices(j):
            i, k = j // K, j % K
            return (pl.ds(i * _WIN, _WIN), pl.ds(k * D, D))

        def wb_start(s, j):
            pltpu.async_copy(row_bufs[s], o_hbm.at[_wb_slices(j)], wsem[s])

        def wb_wait(s, j):
            pltpu.make_async_copy(row_bufs[s], o_hbm.at[_wb_slices(j)],
                                  wsem[s]).wait()

        @pl.when(wid < J)
        def _():
            load_and_gather(0, wid)

        @pl.when(wid + _NW < J)
        def _():
            load_and_gather(1, wid + _NW)

        @pl.loop(0, (Tmax + 1) // 2)
        def _(t):
            jA = wid + _NW * (2 * t)
            jB = jA + _NW
            jC = jB + _NW
            jD = jC + _NW

            @pl.when(jA < J)
            def _():
                gather_wait(0)
                wb_start(0, jA)

            @pl.when(jB < J)
            def _():
                gather_wait(1)
                wb_start(1, jB)

            @pl.when(jC < J)
            def _():
                wb_wait(0, jA)
                load_and_gather(0, jC)

            @pl.when(jD < J)
            def _():
                wb_wait(1, jB)
                load_and_gather(1, jD)

        @pl.when(wid < J)
        def _():
            wb_wait(0, 0)

        @pl.when(wid + _NW < J)
        def _():
            wb_wait(1, 0)

    return gather_kernel(x, idxT)


def _tc_matmul(g, Wb, b2, block_m, rows):
    K = g.shape[1]
    F = Wb.shape[1]

    def mm_kernel(g_ref, w_ref, b_ref, o_ref):
        gb = g_ref[...].astype(jnp.bfloat16)
        o_ref[...] = (
            jnp.dot(gb, w_ref[...], preferred_element_type=jnp.float32)
            + b_ref[...]
        )

    return pl.pallas_call(
        mm_kernel,
        grid=(rows // block_m,),
        in_specs=[
            pl.BlockSpec((block_m, K), lambda i: (i, 0)),
            pl.BlockSpec((K, F), lambda i: (0, 0)),
            pl.BlockSpec((1, F), lambda i: (0, 0)),
        ],
        out_specs=pl.BlockSpec((block_m, F), lambda i: (i, 0)),
        out_shape=jax.ShapeDtypeStruct((rows, F), jnp.float32),
    )(g, Wb, b2)


def kernel(x, hex_in, W, b):
    N, D = x.shape
    Np = (N + 127) // 128 * 128
    idxT = jnp.pad(hex_in.astype(jnp.int32).T, ((0, 0), (0, Np - N)))
    Wb = W.astype(jnp.bfloat16)
    b2 = b.reshape(1, -1)
    g = _sc_gather_wide(x, idxT)
    return _tc_matmul(g, Wb, b2, block_m=400, rows=N)


# speedup vs baseline: 1.3011x; 1.2442x over previous
import jax
import jax.numpy as jnp
from jax import lax
from jax.experimental import pallas as pl
from jax.experimental.pallas import tpu as pltpu
from jax.experimental.pallas import tpu_sc as plsc

_WIN = 128
_NW = 32


def _sc_gather_wide(x, idxT):
    K, Npc = idxT.shape
    D = x.shape[1]
    nwin = Npc // _WIN
    J = K * nwin
    Tmax = (J + _NW - 1) // _NW
    mesh = plsc.VectorSubcoreMesh(core_axis_name="core",
                                  subcore_axis_name="subcore")

    @pl.kernel(
        out_type=jax.ShapeDtypeStruct((Npc, K * D), x.dtype),
        mesh=mesh,
        scratch_types=[
            pltpu.VMEM((1, _WIN), jnp.int32),
            pltpu.VMEM((1, _WIN), jnp.int32),
            pltpu.VMEM((_WIN, D), x.dtype),
            pltpu.VMEM((_WIN, D), x.dtype),
            pltpu.SemaphoreType.DMA,
            pltpu.SemaphoreType.DMA,
            pltpu.SemaphoreType.DMA,
            pltpu.SemaphoreType.DMA,
        ],
    )
    def gather_kernel(x_hbm, i_hbm, o_hbm, idx0, idx1, rows0, rows1,
                      g0, g1, w0, w1):
        wid = lax.axis_index("core") * 16 + lax.axis_index("subcore")
        idx_bufs = (idx0, idx1)
        row_bufs = (rows0, rows1)
        gsem = (g0, g1)
        wsem = (w0, w1)

        def load_and_gather(s, j):
            i, k = j // K, j % K
            pltpu.sync_copy(i_hbm.at[pl.ds(k, 1), pl.ds(i * _WIN, _WIN)],
                            idx_bufs[s])
            pltpu.async_copy(x_hbm.at[idx_bufs[s].at[0]], row_bufs[s], gsem[s])

        def gather_wait(s):
            pltpu.make_async_copy(x_hbm.at[idx_bufs[s].at[0]], row_bufs[s],
                                  gsem[s]).wait()

        def _wb_slices(j):
            i, k = j // K, j % K
            return (pl.ds(i * _WIN, _WIN), pl.ds(k * D, D))

        def wb_start(s, j):
            pltpu.async_copy(row_bufs[s], o_hbm.at[_wb_slices(j)], wsem[s])

        def wb_wait(s, j):
            pltpu.make_async_copy(row_bufs[s], o_hbm.at[_wb_slices(j)],
                                  wsem[s]).wait()

        @pl.when(wid < J)
        def _():
            load_and_gather(0, wid)

        @pl.when(wid + _NW < J)
        def _():
            load_and_gather(1, wid + _NW)

        @pl.loop(0, (Tmax + 1) // 2)
        def _(t):
            jA = wid + _NW * (2 * t)
            jB = jA + _NW
            jC = jB + _NW
            jD = jC + _NW

            @pl.when(jA < J)
            def _():
                gather_wait(0)
                wb_start(0, jA)

            @pl.when(jB < J)
            def _():
                gather_wait(1)
                wb_start(1, jB)

            @pl.when(jC < J)
            def _():
                wb_wait(0, jA)
                load_and_gather(0, jC)

            @pl.when(jD < J)
            def _():
                wb_wait(1, jB)
                load_and_gather(1, jD)

        @pl.when(wid < J)
        def _():
            wb_wait(0, 0)

        @pl.when(wid + _NW < J)
        def _():
            wb_wait(1, 0)

    return gather_kernel(x, idxT)


def _tc_matmul(g, Wb, b2, block_m, rows):
    K = g.shape[1]
    F = Wb.shape[1]

    def mm_kernel(g_ref, w_ref, b_ref, o_ref):
        gb = g_ref[...].astype(jnp.bfloat16)
        o_ref[...] = (
            jnp.dot(gb, w_ref[...], preferred_element_type=jnp.float32)
            + b_ref[...]
        )

    return pl.pallas_call(
        mm_kernel,
        grid=(rows // block_m,),
        in_specs=[
            pl.BlockSpec((block_m, K), lambda i: (i, 0)),
            pl.BlockSpec((K, F), lambda i: (0, 0)),
            pl.BlockSpec((1, F), lambda i: (0, 0)),
        ],
        out_specs=pl.BlockSpec((block_m, F), lambda i: (i, 0)),
        out_shape=jax.ShapeDtypeStruct((rows, F), jnp.float32),
    )(g, Wb, b2)


def kernel(x, hex_in, W, b):
    N, D = x.shape
    Np = (N + 127) // 128 * 128
    idxT = jnp.pad(hex_in.astype(jnp.int32).T, ((0, 0), (0, Np - N)))
    Wb = W.astype(jnp.bfloat16)
    b2 = b.reshape(1, -1)
    g = _sc_gather_wide(x, idxT)
    return _tc_matmul(g, Wb, b2, block_m=2000, rows=N)


# speedup vs baseline: 1.3044x; 1.0025x over previous
import jax
import jax.numpy as jnp
from jax import lax
from jax.experimental import pallas as pl
from jax.experimental.pallas import tpu as pltpu
from jax.experimental.pallas import tpu_sc as plsc

_WIN = 128
_NW = 32


def _sc_gather_wide(x, idxT):
    K, Npc = idxT.shape
    D = x.shape[1]
    nwin = Npc // _WIN
    J = K * nwin
    Tmax = (J + _NW - 1) // _NW
    mesh = plsc.VectorSubcoreMesh(core_axis_name="core",
                                  subcore_axis_name="subcore")

    @pl.kernel(
        out_type=jax.ShapeDtypeStruct((Npc, K * D), x.dtype),
        mesh=mesh,
        scratch_types=[
            pltpu.VMEM((1, _WIN), jnp.int32),
            pltpu.VMEM((1, _WIN), jnp.int32),
            pltpu.VMEM((_WIN, D), x.dtype),
            pltpu.VMEM((_WIN, D), x.dtype),
            pltpu.SemaphoreType.DMA,
            pltpu.SemaphoreType.DMA,
            pltpu.SemaphoreType.DMA,
            pltpu.SemaphoreType.DMA,
        ],
    )
    def gather_kernel(x_hbm, i_hbm, o_hbm, idx0, idx1, rows0, rows1,
                      g0, g1, w0, w1):
        wid = lax.axis_index("core") * 16 + lax.axis_index("subcore")
        idx_bufs = (idx0, idx1)
        row_bufs = (rows0, rows1)
        gsem = (g0, g1)
        wsem = (w0, w1)

        def load_and_gather(s, j):
            i, k = j // K, j % K
            pltpu.sync_copy(i_hbm.at[pl.ds(k, 1), pl.ds(i * _WIN, _WIN)],
                            idx_bufs[s])
            pltpu.async_copy(x_hbm.at[idx_bufs[s].at[0]], row_bufs[s], gsem[s])

        def gather_wait(s):
            pltpu.make_async_copy(x_hbm.at[idx_bufs[s].at[0]], row_bufs[s],
                                  gsem[s]).wait()

        def _wb_slices(j):
            i, k = j // K, j % K
            return (pl.ds(i * _WIN, _WIN), pl.ds(k * D, D))

        def wb_start(s, j):
            pltpu.async_copy(row_bufs[s], o_hbm.at[_wb_slices(j)], wsem[s])

        def wb_wait(s, j):
            pltpu.make_async_copy(row_bufs[s], o_hbm.at[_wb_slices(j)],
                                  wsem[s]).wait()

        @pl.when(wid < J)
        def _():
            load_and_gather(0, wid)

        @pl.when(wid + _NW < J)
        def _():
            load_and_gather(1, wid + _NW)

        @pl.loop(0, (Tmax + 1) // 2)
        def _(t):
            jA = wid + _NW * (2 * t)
            jB = jA + _NW
            jC = jB + _NW
            jD = jC + _NW

            @pl.when(jA < J)
            def _():
                gather_wait(0)
                wb_start(0, jA)

            @pl.when(jB < J)
            def _():
                gather_wait(1)
                wb_start(1, jB)

            @pl.when(jC < J)
            def _():
                wb_wait(0, jA)
                load_and_gather(0, jC)

            @pl.when(jD < J)
            def _():
                wb_wait(1, jB)
                load_and_gather(1, jD)

        @pl.when(wid < J)
        def _():
            wb_wait(0, 0)

        @pl.when(wid + _NW < J)
        def _():
            wb_wait(1, 0)

    return gather_kernel(x, idxT)


def _tc_matmul(g, Wb, b2, block_m, rows):
    K = g.shape[1]
    F = Wb.shape[1]

    def mm_kernel(g_ref, w_ref, b_ref, o_ref):
        gb = g_ref[...].astype(jnp.bfloat16)
        o_ref[...] = (
            jnp.dot(gb, w_ref[...], preferred_element_type=jnp.float32)
            + b_ref[...]
        )

    return pl.pallas_call(
        mm_kernel,
        grid=(rows // block_m,),
        in_specs=[
            pl.BlockSpec((block_m, K), lambda i: (i, 0)),
            pl.BlockSpec((K, F), lambda i: (0, 0)),
            pl.BlockSpec((1, F), lambda i: (0, 0)),
        ],
        out_specs=pl.BlockSpec((block_m, F), lambda i: (i, 0)),
        out_shape=jax.ShapeDtypeStruct((rows, F), jnp.float32),
    )(g, Wb, b2)


def kernel(x, hex_in, W, b):
    N, D = x.shape
    Np = (N + 127) // 128 * 128
    idxT = jnp.pad(hex_in.astype(jnp.int32).T, ((0, 0), (0, Np - N)))
    Wb = W.astype(jnp.bfloat16)
    b2 = b.reshape(1, -1)
    g = _sc_gather_wide(x, idxT)
    return _tc_matmul(g, Wb, b2, block_m=5000, rows=N)


# speedup vs baseline: 1.3931x; 1.0680x over previous
import jax
import jax.numpy as jnp
from jax import lax
from jax.experimental import pallas as pl
from jax.experimental.pallas import tpu as pltpu
from jax.experimental.pallas import tpu_sc as plsc

_WIN = 128
_NW = 32


def _sc_gather_wide(x, idxT):
    K, Npc = idxT.shape
    D = x.shape[1]
    nwin = Npc // _WIN
    J = K * nwin
    Tmax = (J + _NW - 1) // _NW
    mesh = plsc.VectorSubcoreMesh(core_axis_name="core",
                                  subcore_axis_name="subcore")

    nslot = 4
    @pl.kernel(
        out_type=jax.ShapeDtypeStruct((Npc, K * D), x.dtype),
        mesh=mesh,
        scratch_types=(
            [pltpu.VMEM((1, _WIN), jnp.int32)] * nslot
            + [pltpu.VMEM((_WIN, D), x.dtype)] * nslot
            + [pltpu.SemaphoreType.DMA] * (2 * nslot)
        ),
    )
    def gather_kernel(x_hbm, i_hbm, o_hbm, *scratch):
        wid = lax.axis_index("core") * 16 + lax.axis_index("subcore")
        idx_bufs = scratch[:nslot]
        row_bufs = scratch[nslot:2 * nslot]
        gsem = scratch[2 * nslot:3 * nslot]
        wsem = scratch[3 * nslot:4 * nslot]

        def load_and_gather(s, j):
            i, k = j // K, j % K
            pltpu.sync_copy(i_hbm.at[pl.ds(k, 1), pl.ds(i * _WIN, _WIN)],
                            idx_bufs[s])
            pltpu.async_copy(x_hbm.at[idx_bufs[s].at[0]], row_bufs[s], gsem[s])

        def gather_wait(s):
            pltpu.make_async_copy(x_hbm.at[idx_bufs[s].at[0]], row_bufs[s],
                                  gsem[s]).wait()

        def _wb_slices(j):
            i, k = j // K, j % K
            return (pl.ds(i * _WIN, _WIN), pl.ds(k * D, D))

        def wb_start(s, j):
            pltpu.async_copy(row_bufs[s], o_hbm.at[_wb_slices(j)], wsem[s])

        def wb_wait(s, j):
            pltpu.make_async_copy(row_bufs[s], o_hbm.at[_wb_slices(j)],
                                  wsem[s]).wait()

        for s in range(nslot):
            @pl.when(wid + _NW * s < J)
            def _(s=s):
                load_and_gather(s, wid + _NW * s)

        @pl.loop(0, (Tmax + nslot - 1) // nslot)
        def _(t):
            base = wid + _NW * nslot * t
            for s in range(nslot):
                @pl.when(base + _NW * s < J)
                def _(s=s):
                    gather_wait(s)
                    wb_start(s, base + _NW * s)
            for s in range(nslot):
                @pl.when(base + _NW * (nslot + s) < J)
                def _(s=s):
                    wb_wait(s, 0)
                    load_and_gather(s, base + _NW * (nslot + s))

        for s in range(nslot):
            @pl.when(wid + _NW * s < J)
            def _(s=s):
                wb_wait(s, 0)

    return gather_kernel(x, idxT)


def _tc_matmul(g, Wb, b2, block_m, rows):
    K = g.shape[1]
    F = Wb.shape[1]

    def mm_kernel(g_ref, w_ref, b_ref, o_ref):
        gb = g_ref[...].astype(jnp.bfloat16)
        o_ref[...] = (
            jnp.dot(gb, w_ref[...], preferred_element_type=jnp.float32)
            + b_ref[...]
        )

    return pl.pallas_call(
        mm_kernel,
        grid=(rows // block_m,),
        in_specs=[
            pl.BlockSpec((block_m, K), lambda i: (i, 0)),
            pl.BlockSpec((K, F), lambda i: (0, 0)),
            pl.BlockSpec((1, F), lambda i: (0, 0)),
        ],
        out_specs=pl.BlockSpec((block_m, F), lambda i: (i, 0)),
        out_shape=jax.ShapeDtypeStruct((rows, F), jnp.float32),
    )(g, Wb, b2)


def kernel(x, hex_in, W, b):
    N, D = x.shape
    Np = (N + 127) // 128 * 128
    idxT = jnp.pad(hex_in.astype(jnp.int32).T, ((0, 0), (0, Np - N)))
    Wb = W.astype(jnp.bfloat16)
    b2 = b.reshape(1, -1)
    g = _sc_gather_wide(x, idxT)
    return _tc_matmul(g, Wb, b2, block_m=5000, rows=N)


# speedup vs baseline: 1.4039x; 1.0077x over previous
import jax
import jax.numpy as jnp
from jax import lax
from jax.experimental import pallas as pl
from jax.experimental.pallas import tpu as pltpu
from jax.experimental.pallas import tpu_sc as plsc

_WIN = 128
_NW = 32


def _sc_gather_wide(x, idxT):
    K, Npc = idxT.shape
    D = x.shape[1]
    nwin = Npc // _WIN
    J = K * nwin
    Tmax = (J + _NW - 1) // _NW
    mesh = plsc.VectorSubcoreMesh(core_axis_name="core",
                                  subcore_axis_name="subcore")

    nslot = 6
    @pl.kernel(
        out_type=jax.ShapeDtypeStruct((Npc, K * D), x.dtype),
        mesh=mesh,
        scratch_types=(
            [pltpu.VMEM((1, _WIN), jnp.int32)] * nslot
            + [pltpu.VMEM((_WIN, D), x.dtype)] * nslot
            + [pltpu.SemaphoreType.DMA] * (2 * nslot)
        ),
    )
    def gather_kernel(x_hbm, i_hbm, o_hbm, *scratch):
        wid = lax.axis_index("core") * 16 + lax.axis_index("subcore")
        idx_bufs = scratch[:nslot]
        row_bufs = scratch[nslot:2 * nslot]
        gsem = scratch[2 * nslot:3 * nslot]
        wsem = scratch[3 * nslot:4 * nslot]

        def load_and_gather(s, j):
            i, k = j // K, j % K
            pltpu.sync_copy(i_hbm.at[pl.ds(k, 1), pl.ds(i * _WIN, _WIN)],
                            idx_bufs[s])
            pltpu.async_copy(x_hbm.at[idx_bufs[s].at[0]], row_bufs[s], gsem[s])

        def gather_wait(s):
            pltpu.make_async_copy(x_hbm.at[idx_bufs[s].at[0]], row_bufs[s],
                                  gsem[s]).wait()

        def _wb_slices(j):
            i, k = j // K, j % K
            return (pl.ds(i * _WIN, _WIN), pl.ds(k * D, D))

        def wb_start(s, j):
            pltpu.async_copy(row_bufs[s], o_hbm.at[_wb_slices(j)], wsem[s])

        def wb_wait(s, j):
            pltpu.make_async_copy(row_bufs[s], o_hbm.at[_wb_slices(j)],
                                  wsem[s]).wait()

        for s in range(nslot):
            @pl.when(wid + _NW * s < J)
            def _(s=s):
                load_and_gather(s, wid + _NW * s)

        @pl.loop(0, (Tmax + nslot - 1) // nslot)
        def _(t):
            base = wid + _NW * nslot * t
            for s in range(nslot):
                @pl.when(base + _NW * s < J)
                def _(s=s):
                    gather_wait(s)
                    wb_start(s, base + _NW * s)
            for s in range(nslot):
                @pl.when(base + _NW * (nslot + s) < J)
                def _(s=s):
                    wb_wait(s, 0)
                    load_and_gather(s, base + _NW * (nslot + s))

        for s in range(nslot):
            @pl.when(wid + _NW * s < J)
            def _(s=s):
                wb_wait(s, 0)

    return gather_kernel(x, idxT)


def _tc_matmul(g, Wb, b2, block_m, rows):
    K = g.shape[1]
    F = Wb.shape[1]

    def mm_kernel(g_ref, w_ref, b_ref, o_ref):
        gb = g_ref[...].astype(jnp.bfloat16)
        o_ref[...] = (
            jnp.dot(gb, w_ref[...], preferred_element_type=jnp.float32)
            + b_ref[...]
        )

    return pl.pallas_call(
        mm_kernel,
        grid=(rows // block_m,),
        in_specs=[
            pl.BlockSpec((block_m, K), lambda i: (i, 0)),
            pl.BlockSpec((K, F), lambda i: (0, 0)),
            pl.BlockSpec((1, F), lambda i: (0, 0)),
        ],
        out_specs=pl.BlockSpec((block_m, F), lambda i: (i, 0)),
        out_shape=jax.ShapeDtypeStruct((rows, F), jnp.float32),
    )(g, Wb, b2)


def kernel(x, hex_in, W, b):
    N, D = x.shape
    Np = (N + 127) // 128 * 128
    idxT = jnp.pad(hex_in.astype(jnp.int32).T, ((0, 0), (0, Np - N)))
    Wb = W.astype(jnp.bfloat16)
    b2 = b.reshape(1, -1)
    g = _sc_gather_wide(x, idxT)
    return _tc_matmul(g, Wb, b2, block_m=5000, rows=N)
